# baseline (speedup 1.0000x reference)
import functools

import jax
import jax.numpy as jnp
from jax import lax
from jax.experimental import pallas as pl

Q_OUT = 2400
V_OUT = 768
F_IN = 1200
F_HID = 256
TOTAL = 95
NUM_ANS = {0: 2, 1: 2, 2: 4, 3: 89}
IDXS = {0: [0, 1], 1: [0, 1], 2: list(range(2, 6)), 3: list(range(6, 95))}

BM = 512


def _bf16_dot(a, b):
    return jax.lax.dot_general(
        a.astype(jnp.bfloat16), b.astype(jnp.bfloat16),
        (((1,), (0,)), ((), ())),
        preferred_element_type=jnp.float32)


def _fused_kernel(iv_ref, iq_ref, qt_ref, wv_ref, bv_ref, wq_ref, bq_ref,
                  w1_ref, b1_ref, w2_ref, b2_ref, out_ref):
    xv = jnp.tanh(_bf16_dot(iv_ref[...], wv_ref[...]) + bv_ref[...])
    xq = jnp.tanh(_bf16_dot(iq_ref[...], wq_ref[...]) + bq_ref[...])
    x = jnp.tanh(xv * xq)
    h = jnp.tanh(_bf16_dot(x, w1_ref[...]) + b1_ref[...])
    qt = qt_ref[...]
    blk = lax.broadcasted_iota(jnp.int32, (BM, 4 * F_HID), 1) // F_HID
    h_m = jnp.where(blk == qt, h, 0.0)
    out = _bf16_dot(h_m, w2_ref[...])
    b2 = b2_ref[...]
    for t in range(4):
        out = out + jnp.where(qt == t, 1.0, 0.0) * b2[t][None, :]
    out_ref[...] = out


@functools.partial(jax.jit, static_argnames=())
def kernel(input_v, input_q, question_type, W_v, b_v, W_q, b_q, cls_params):
    n = input_v.shape[0]
    qt = question_type.astype(jnp.int32).reshape(n, 1)

    W1_all = jnp.concatenate([cls_params[t][0] for t in range(4)], axis=1)
    b1_all = jnp.concatenate([cls_params[t][1] for t in range(4)], axis=0)
    w2_cols = []
    b2_rows = []
    for t in range(4):
        W2, b2 = cls_params[t][2], cls_params[t][3]
        idx = jnp.asarray(IDXS[t], dtype=jnp.int32)
        w2_cols.append(jnp.zeros((F_HID, TOTAL), jnp.float32).at[:, idx].set(W2))
        b2_rows.append(jnp.zeros((TOTAL,), jnp.float32).at[idx].set(b2))
    W2_all = jnp.concatenate(w2_cols, axis=0)
    B2_rows = jnp.stack(b2_rows + [jnp.zeros((TOTAL,), jnp.float32)] * 4)

    grid = (n // BM,)
    out = pl.pallas_call(
        _fused_kernel,
        grid=grid,
        in_specs=[
            pl.BlockSpec((BM, V_OUT), lambda i: (i, 0)),
            pl.BlockSpec((BM, Q_OUT), lambda i: (i, 0)),
            pl.BlockSpec((BM, 1), lambda i: (i, 0)),
            pl.BlockSpec((V_OUT, F_IN), lambda i: (0, 0)),
            pl.BlockSpec((1, F_IN), lambda i: (0, 0)),
            pl.BlockSpec((Q_OUT, F_IN), lambda i: (0, 0)),
            pl.BlockSpec((1, F_IN), lambda i: (0, 0)),
            pl.BlockSpec((F_IN, 4 * F_HID), lambda i: (0, 0)),
            pl.BlockSpec((1, 4 * F_HID), lambda i: (0, 0)),
            pl.BlockSpec((4 * F_HID, TOTAL), lambda i: (0, 0)),
            pl.BlockSpec((8, TOTAL), lambda i: (0, 0)),
        ],
        out_specs=pl.BlockSpec((BM, TOTAL), lambda i: (i, 0)),
        out_shape=jax.ShapeDtypeStruct((n, TOTAL), jnp.float32),
    )(input_v, input_q, qt, W_v, b_v.reshape(1, F_IN), W_q,
      b_q.reshape(1, F_IN), W1_all, b1_all.reshape(1, 4 * F_HID),
      W2_all, B2_rows)
    return out


# speedup vs baseline: 1.0138x; 1.0138x over previous
import functools

import jax
import jax.numpy as jnp
from jax import lax
from jax.experimental import pallas as pl

Q_OUT = 2400
V_OUT = 768
F_IN = 1200
F_HID = 256
TOTAL = 95
NUM_ANS = {0: 2, 1: 2, 2: 4, 3: 89}
IDXS = {0: [0, 1], 1: [0, 1], 2: list(range(2, 6)), 3: list(range(6, 95))}

BM = 1024


def _dot(a, b):
    return jax.lax.dot_general(
        a.astype(jnp.bfloat16), b,
        (((1,), (0,)), ((), ())),
        preferred_element_type=jnp.float32)


def _fused_kernel(iv_ref, iq_ref, qt_ref, wv_ref, bv_ref, wq_ref, bq_ref,
                  w1_ref, b1_ref, w2_ref, b2_ref, out_ref):
    xv = jnp.tanh(_dot(iv_ref[...], wv_ref[...]) + bv_ref[...])
    xq = jnp.tanh(_dot(iq_ref[...], wq_ref[...]) + bq_ref[...])
    x = jnp.tanh(xv * xq)
    h = jnp.tanh(_dot(x, w1_ref[...]) + b1_ref[...])
    qt = qt_ref[...]
    blk = lax.broadcasted_iota(jnp.int32, (BM, 4 * F_HID), 1) // F_HID
    h_m = jnp.where(blk == qt, h, 0.0)
    out = _dot(h_m, w2_ref[...])
    b2 = b2_ref[...]
    for t in range(4):
        out = out + jnp.where(qt == t, 1.0, 0.0) * b2[t][None, :]
    out_ref[...] = out


@functools.partial(jax.jit, static_argnames=())
def kernel(input_v, input_q, question_type, W_v, b_v, W_q, b_q, cls_params):
    n = input_v.shape[0]
    qt = question_type.astype(jnp.int32).reshape(n, 1)

    bf = jnp.bfloat16
    W1_all = jnp.concatenate(
        [cls_params[t][0].astype(bf) for t in range(4)], axis=1)
    b1_all = jnp.concatenate([cls_params[t][1] for t in range(4)], axis=0)
    w2_cols = []
    b2_rows = []
    for t in range(4):
        W2, b2 = cls_params[t][2], cls_params[t][3]
        idx = jnp.asarray(IDXS[t], dtype=jnp.int32)
        w2_cols.append(jnp.zeros((F_HID, TOTAL), bf).at[:, idx].set(W2.astype(bf)))
        b2_rows.append(jnp.zeros((TOTAL,), jnp.float32).at[idx].set(b2))
    W2_all = jnp.concatenate(w2_cols, axis=0)
    B2_rows = jnp.stack(b2_rows + [jnp.zeros((TOTAL,), jnp.float32)] * 4)

    grid = (n // BM,)
    out = pl.pallas_call(
        _fused_kernel,
        grid=grid,
        in_specs=[
            pl.BlockSpec((BM, V_OUT), lambda i: (i, 0)),
            pl.BlockSpec((BM, Q_OUT), lambda i: (i, 0)),
            pl.BlockSpec((BM, 1), lambda i: (i, 0)),
            pl.BlockSpec((V_OUT, F_IN), lambda i: (0, 0)),
            pl.BlockSpec((1, F_IN), lambda i: (0, 0)),
            pl.BlockSpec((Q_OUT, F_IN), lambda i: (0, 0)),
            pl.BlockSpec((1, F_IN), lambda i: (0, 0)),
            pl.BlockSpec((F_IN, 4 * F_HID), lambda i: (0, 0)),
            pl.BlockSpec((1, 4 * F_HID), lambda i: (0, 0)),
            pl.BlockSpec((4 * F_HID, TOTAL), lambda i: (0, 0)),
            pl.BlockSpec((8, TOTAL), lambda i: (0, 0)),
        ],
        out_specs=pl.BlockSpec((BM, TOTAL), lambda i: (i, 0)),
        out_shape=jax.ShapeDtypeStruct((n, TOTAL), jnp.float32),
    )(input_v, input_q, qt, W_v.astype(bf), b_v.reshape(1, F_IN),
      W_q.astype(bf), b_q.reshape(1, F_IN), W1_all,
      b1_all.reshape(1, 4 * F_HID), W2_all, B2_rows)
    return out
